# baseline (speedup 1.0000x reference)
import jax
import jax.numpy as jnp
from jax import lax
from jax.experimental import pallas as pl
from jax.experimental.pallas import tpu as pltpu
from jax.experimental.pallas import tpu_sc as plsc

N = 10000
E = 320000
F = 128
NCLASS = 64

NC = 2
NS = 16
NW = NC * NS
CHUNK = 96
NCHUNK = 105
EPW = CHUNK * NCHUNK
EPAD = NW * EPW
NSLOT = 4
NITER = NCHUNK // NSLOT
ZB = 400
NZB = N // ZB


def _spmm_body(sup_hbm, src_hbm, dst_hbm, ew_hbm, zeros_hbm, out_hbm,
               *scr):
    srcv = scr[0:4]
    dstv = scr[4:8]
    ewv = scr[8:12]
    rows = scr[12:16]
    acc_sh = scr[16]
    isem = scr[17:21]
    dsem = scr[21:25]
    gsem = scr[25:29]
    ssem = scr[29:33]

    cid = lax.axis_index("c")
    sid = lax.axis_index("s")
    wid = cid * NS + sid

    for k in range((NZB + NS - 1) // NS):
        blk = k * NS + sid
        @pl.when(blk < NZB)
        def _():
            r0 = pl.multiple_of(blk * ZB, 8)
            pltpu.sync_copy(zeros_hbm.at[pl.ds(r0, ZB)],
                            acc_sh.at[pl.ds(r0, ZB)])
    plsc.subcore_barrier()

    def istart(c, s):
        pltpu.async_copy(src_hbm.at[wid, c, 0], srcv[s], isem[s])
        pltpu.async_copy(ew_hbm.at[wid, c, 0], ewv[s], isem[s])

    def iwait(c, s):
        pltpu.make_async_copy(src_hbm.at[wid, c, 0], srcv[s], isem[s]).wait()
        pltpu.make_async_copy(ew_hbm.at[wid, c, 0], ewv[s], isem[s]).wait()

    def dstart(c, s):
        pltpu.async_copy(dst_hbm.at[wid, c, 0], dstv[s], dsem[s])

    def dwait(c, s):
        pltpu.make_async_copy(dst_hbm.at[wid, c, 0], dstv[s], dsem[s]).wait()

    def gstart(s):
        pltpu.async_copy(sup_hbm.at[srcv[s]], rows[s], gsem[s])

    def gwait(s):
        pltpu.make_async_copy(sup_hbm.at[srcv[s]], rows[s], gsem[s]).wait()

    def sstart(s):
        pltpu.async_copy(rows[s], acc_sh.at[dstv[s]], ssem[s], add=True)

    def swait(s):
        pltpu.make_async_copy(rows[s], acc_sh.at[dstv[s]], ssem[s]).wait()

    def mult(s):
        ew_v, rows_v = ewv[s], rows[s]

        def group(g, c2):
            ew16 = ew_v[pl.ds(g * 16, 16)]
            for t in range(16):
                wv = jnp.full((16,), ew16[t], jnp.float32)
                e = g * 16 + t
                for f in range(F // 16):
                    sl = pl.ds(f * 16, 16)
                    rows_v[e, sl] = rows_v[e, sl] * wv
            return c2

        lax.fori_loop(0, CHUNK // 16, group, 0, unroll=False)

    def chunk_step(c, j):
        gwait(j)
        mult(j)
        dwait(c, j)
        sstart(j)

        @pl.when(c + 2 < NCHUNK)
        def _():
            s2 = (j + 2) % NSLOT
            @pl.when(c >= 2)
            def _():
                swait(s2)
            iwait(c + 2, s2)
            dstart(c + 2, s2)
            gstart(s2)

        @pl.when(c + 4 < NCHUNK)
        def _():
            istart(c + 4, j)

    for c0 in range(NSLOT):
        istart(c0, c0)
    dstart(0, 0)
    dstart(1, 1)
    iwait(0, 0)
    gstart(0)
    iwait(1, 1)
    gstart(1)

    def ring_body(k, carry):
        c = k * NSLOT
        for j in range(NSLOT):
            chunk_step(c + j, j)
        return carry

    lax.fori_loop(0, NITER, ring_body, 0, unroll=False)

    for c in range(NITER * NSLOT, NCHUNK):
        chunk_step(jnp.int32(c), c % NSLOT)
    for c in range(NCHUNK - NSLOT, NCHUNK):
        swait(c % NSLOT)

    plsc.subcore_barrier()
    for k in range((NZB + NS - 1) // NS):
        blk = k * NS + sid
        @pl.when(blk < NZB)
        def _():
            r0 = pl.multiple_of(blk * ZB, 8)
            pltpu.sync_copy(acc_sh.at[pl.ds(r0, ZB)],
                            out_hbm.at[cid, pl.ds(r0, ZB)])


_spmm = pl.kernel(
    _spmm_body,
    out_type=jax.ShapeDtypeStruct((NC, N, F), jnp.float32),
    mesh=plsc.VectorSubcoreMesh(core_axis_name="c", subcore_axis_name="s",
                                num_cores=NC, num_subcores=NS),
    scratch_types=(
        [pltpu.VMEM((CHUNK,), jnp.int32) for _ in range(4)]
        + [pltpu.VMEM((CHUNK,), jnp.int32) for _ in range(4)]
        + [pltpu.VMEM((CHUNK,), jnp.float32) for _ in range(4)]
        + [pltpu.VMEM((CHUNK, F), jnp.float32) for _ in range(4)]
        + [pltpu.VMEM_SHARED((N, F), jnp.float32)]
        + [pltpu.SemaphoreType.DMA for _ in range(16)]
    ),
)


def _mm_kernel(x_ref, w_ref, o_ref):
    o_ref[...] = jnp.dot(x_ref[...], w_ref[...],
                         preferred_element_type=jnp.float32)


def _fuse_kernel(p_ref, b_ref, w_ref, o_ref):
    h = jnp.maximum(p_ref[0] + p_ref[1] + b_ref[...], 0.0)
    o_ref[...] = jnp.dot(h, w_ref[...], preferred_element_type=jnp.float32)


def _heads_kernel(p_ref, b_ref, w_ref, hb_ref, o_ref):
    h = jnp.maximum(p_ref[0] + p_ref[1] + b_ref[...], 0.0)
    o_ref[...] = jnp.dot(h, w_ref[...],
                         preferred_element_type=jnp.float32) + hb_ref[...]


def kernel(x, edge_index, edge_weight, W1, b1, W2, b2, L1W, L1b, L2W, L2b):
    pad = EPAD - E
    src3 = jnp.concatenate(
        [edge_index[0], jnp.zeros((pad,), jnp.int32)]
    ).reshape(NW, NCHUNK, 1, CHUNK)
    dst3 = jnp.concatenate(
        [edge_index[1], jnp.zeros((pad,), jnp.int32)]
    ).reshape(NW, NCHUNK, 1, CHUNK)
    ew3 = jnp.concatenate(
        [edge_weight, jnp.zeros((pad,), jnp.float32)]
    ).reshape(NW, NCHUNK, 1, CHUNK)
    zeros = jnp.zeros((N, F), jnp.float32)

    support1 = pl.pallas_call(
        _mm_kernel,
        out_shape=jax.ShapeDtypeStruct((N, F), jnp.float32),
    )(x, W1)

    p = _spmm(support1, src3, dst3, ew3, zeros)

    support2 = pl.pallas_call(
        _fuse_kernel,
        out_shape=jax.ShapeDtypeStruct((N, F), jnp.float32),
    )(p, b1.reshape(1, F), W2)

    q = _spmm(support2, src3, dst3, ew3, zeros)

    Wcat = jnp.concatenate([L1W, L2W], axis=1)
    bcat = jnp.concatenate([L1b, L2b]).reshape(1, 2 * NCLASS)
    out = pl.pallas_call(
        _heads_kernel,
        out_shape=jax.ShapeDtypeStruct((N, 2 * NCLASS), jnp.float32),
    )(q, b2.reshape(1, F), Wcat, bcat)

    return out[:, :NCLASS], out[:, NCLASS:]
